# baseline (speedup 1.0000x reference)
import functools

import numpy as np
import jax
import jax.numpy as jnp
from jax.experimental import pallas as pl
from jax.experimental.pallas import tpu as pltpu

N_BINS = 15
_BOUNDS = np.linspace(0.0, 1.0, N_BINS + 1)


def _ece_tc_kernel(n_total, logits_ref, labels_ref, bounds_ref, cnt_ref,
                   sc_ref, sa_ref, ece_ref):
    i = pl.program_id(0)
    nsteps = pl.num_programs(0)
    r, g, c = logits_ref.shape

    @pl.when(i == 0)
    def _init():
        cnt_ref[...] = jnp.zeros_like(cnt_ref)
        sc_ref[...] = jnp.zeros_like(sc_ref)
        sa_ref[...] = jnp.zeros_like(sa_ref)

    x = logits_ref[...]
    lab = labels_ref[...].astype(jnp.float32)
    m = jnp.max(x, axis=2, keepdims=True)
    s = jnp.sum(jnp.exp(x - m), axis=2)
    conf = 1.0 / s
    iota_f = jax.lax.broadcasted_iota(jnp.int32, (r, g, c), 2).astype(
        jnp.float32)
    pred = jnp.min(jnp.where(x == m, iota_f, np.float32(c)),
                   axis=2)
    acc = (pred == lab).astype(jnp.float32)

    lo = bounds_ref[0:1, :].reshape(1, 1, N_BINS)
    hi = bounds_ref[1:2, :].reshape(1, 1, N_BINS)
    conf3 = conf[:, :, None]
    mask = (conf3 > lo) & (conf3 <= hi)
    m_cnt = jnp.where(mask, 1.0, 0.0)
    m_sc = jnp.where(mask, conf3, 0.0)
    m_sa = jnp.where(mask, acc[:, :, None], 0.0)
    cnt_ref[...] += jnp.sum(m_cnt, axis=(0, 1)).reshape(1, N_BINS)
    sc_ref[...] += jnp.sum(m_sc, axis=(0, 1)).reshape(1, N_BINS)
    sa_ref[...] += jnp.sum(m_sa, axis=(0, 1)).reshape(1, N_BINS)

    @pl.when(i == nsteps - 1)
    def _finish():
        cnt = cnt_ref[...]
        safe = jnp.maximum(cnt, 1.0)
        avg_conf = sc_ref[...] / safe
        avg_acc = sa_ref[...] / safe
        prop = cnt / np.float32(n_total)
        contrib = jnp.abs(avg_conf - avg_acc) * prop
        ece_ref[...] = jnp.sum(jnp.where(cnt > 0, contrib, 0.0),
                               keepdims=True)


def kernel(logits, labels):
    n, c = logits.shape
    block = 8000
    assert n % block == 0
    nsteps = n // block
    rows = block // 8
    logits3 = logits.reshape(n // 8, 8, c)
    labels2 = labels.reshape(n // 8, 8)
    bounds = jnp.asarray(
        np.stack([_BOUNDS[:-1], _BOUNDS[1:]]).astype(np.float32))

    body = functools.partial(_ece_tc_kernel, n)
    out = pl.pallas_call(
        body,
        grid=(nsteps,),
        in_specs=[
            pl.BlockSpec((rows, 8, c), lambda i: (i, 0, 0)),
            pl.BlockSpec((rows, 8), lambda i: (i, 0)),
            pl.BlockSpec((2, N_BINS), lambda i: (0, 0)),
        ],
        out_specs=[
            pl.BlockSpec((1, N_BINS), lambda i: (0, 0)),
            pl.BlockSpec((1, N_BINS), lambda i: (0, 0)),
            pl.BlockSpec((1, N_BINS), lambda i: (0, 0)),
            pl.BlockSpec((1, 1), lambda i: (0, 0)),
        ],
        out_shape=[
            jax.ShapeDtypeStruct((1, N_BINS), jnp.float32),
            jax.ShapeDtypeStruct((1, N_BINS), jnp.float32),
            jax.ShapeDtypeStruct((1, N_BINS), jnp.float32),
            jax.ShapeDtypeStruct((1, 1), jnp.float32),
        ],
        compiler_params=pltpu.CompilerParams(
            dimension_semantics=("arbitrary",),
        ),
    )(logits3, labels2, bounds)
    return out[3].reshape(1)


# speedup vs baseline: 2.0581x; 2.0581x over previous
import functools

import numpy as np
import jax
import jax.numpy as jnp
from jax.experimental import pallas as pl
from jax.experimental.pallas import tpu as pltpu

N_BINS = 15
_BOUNDS = np.linspace(0.0, 1.0, N_BINS + 1)


def _ece_tc_kernel(n_total, logits_ref, labels_ref, bounds_ref, cnt_ref,
                   sc_ref, sa_ref, ece_ref):
    i = pl.program_id(0)
    nsteps = pl.num_programs(0)
    b, c = logits_ref.shape

    @pl.when(i == 0)
    def _init():
        cnt_ref[...] = jnp.zeros_like(cnt_ref)
        sc_ref[...] = jnp.zeros_like(sc_ref)
        sa_ref[...] = jnp.zeros_like(sa_ref)

    x = logits_ref[...]
    r = i % 8
    lab = jnp.transpose(labels_ref[pl.ds(r, 1), :], (1, 0))
    m = jnp.max(x, axis=1, keepdims=True)
    s = jnp.sum(jnp.exp(x - m), axis=1, keepdims=True)
    conf = 1.0 / s
    iota_row = jax.lax.broadcasted_iota(jnp.int32, (1, c), 1).astype(
        jnp.float32)
    pred = jnp.min(jnp.where(x == m, iota_row, np.float32(c)),
                   axis=1, keepdims=True)
    acc = (pred == lab).astype(jnp.float32)

    lo = bounds_ref[0:1, :]
    hi = bounds_ref[1:2, :]
    mask = (conf > lo) & (conf <= hi)
    m_cnt = jnp.where(mask, 1.0, 0.0)
    m_sc = jnp.where(mask, conf, 0.0)
    m_sa = jnp.where(mask, acc, 0.0)
    cnt_ref[...] += jnp.sum(m_cnt, axis=0, keepdims=True)
    sc_ref[...] += jnp.sum(m_sc, axis=0, keepdims=True)
    sa_ref[...] += jnp.sum(m_sa, axis=0, keepdims=True)

    @pl.when(i == nsteps - 1)
    def _finish():
        cnt = cnt_ref[...]
        safe = jnp.maximum(cnt, 1.0)
        avg_conf = sc_ref[...] / safe
        avg_acc = sa_ref[...] / safe
        prop = cnt / np.float32(n_total)
        contrib = jnp.abs(avg_conf - avg_acc) * prop
        ece_ref[...] = jnp.sum(jnp.where(cnt > 0, contrib, 0.0),
                               keepdims=True)


def kernel(logits, labels):
    n, c = logits.shape
    block = 8000
    assert n % block == 0
    nsteps = n // block
    labels2 = labels.astype(jnp.float32).reshape(nsteps, block)
    bounds = jnp.asarray(
        np.stack([_BOUNDS[:-1], _BOUNDS[1:]]).astype(np.float32))

    body = functools.partial(_ece_tc_kernel, n)
    out = pl.pallas_call(
        body,
        grid=(nsteps,),
        in_specs=[
            pl.BlockSpec((block, c), lambda i: (i, 0)),
            pl.BlockSpec((8, block), lambda i: (i // 8, 0)),
            pl.BlockSpec((2, N_BINS), lambda i: (0, 0)),
        ],
        out_specs=[
            pl.BlockSpec((1, N_BINS), lambda i: (0, 0)),
            pl.BlockSpec((1, N_BINS), lambda i: (0, 0)),
            pl.BlockSpec((1, N_BINS), lambda i: (0, 0)),
            pl.BlockSpec((1, 1), lambda i: (0, 0)),
        ],
        out_shape=[
            jax.ShapeDtypeStruct((1, N_BINS), jnp.float32),
            jax.ShapeDtypeStruct((1, N_BINS), jnp.float32),
            jax.ShapeDtypeStruct((1, N_BINS), jnp.float32),
            jax.ShapeDtypeStruct((1, 1), jnp.float32),
        ],
        compiler_params=pltpu.CompilerParams(
            dimension_semantics=("arbitrary",),
        ),
    )(logits, labels2, bounds)
    return out[3].reshape(1)
